# baseline (speedup 1.0000x reference)
import jax
import jax.numpy as jnp
from jax import lax
from jax.experimental import pallas as pl
from jax.experimental.pallas import tpu as pltpu
from jax.experimental.pallas import tpu_sc as plsc

_B = 16384
_D = 32
_V = 1_000_000
_NC = 2
_NS = 16
_NW = _NC * _NS
_BPW = _B // _NW
_JT = 7812
_VFULL = _JT * 128
_NTILE = 4 * _JT
_TPW = _NTILE // 16
_FLEN = _NTILE * 1024


def _repack_body(ut3, it3, uf, if_, sem):
    wid = lax.axis_index("s") * _NC + lax.axis_index("c")
    t = wid % 2
    seg = wid // 2

    for tt, src, dst in ((0, ut3, uf), (1, it3, if_)):
        @pl.when(t == tt)
        def _(src=src, dst=dst):
            def body(p):
                i = p // _JT
                j = p % _JT
                cp = pltpu.async_copy(
                    src.at[i].at[:, pl.ds(pl.multiple_of(j * 128, 128), 128)],
                    dst.at[p], sem)
                cp.wait()

            pl.loop(seg * _TPW, (seg + 1) * _TPW)(body)


def _build_repack():
    return pl.kernel(
        _repack_body,
        out_type=(jax.ShapeDtypeStruct((_NTILE, 8, 128), jnp.float32),
                  jax.ShapeDtypeStruct((_NTILE, 8, 128), jnp.float32)),
        mesh=plsc.VectorSubcoreMesh(core_axis_name="c", subcore_axis_name="s",
                                    num_cores=_NC, num_subcores=_NS),
        scratch_types=[
            pltpu.SemaphoreType.DMA,
        ],
    )


def _gather_body(uidx_hbm, iidx_hbm, uf, if_, ut_tail, it_tail, ue2, ie2,
                 uidx_v, iidx_v, eidx_u, eidx_i, urows_v, irows_v,
                 utail_v, itail_v, sem_u, sem_i):
    wid = lax.axis_index("s") * _NC + lax.axis_index("c")
    base = wid * _BPW
    pltpu.sync_copy(uidx_hbm.at[pl.ds(base, _BPW)], uidx_v)
    pltpu.sync_copy(iidx_hbm.at[pl.ds(base, _BPW)], iidx_v)
    pltpu.sync_copy(ut_tail, utail_v)
    pltpu.sync_copy(it_tail, itail_v)

    def chunk(j):
        for idx_v, eidx_v in ((uidx_v, eidx_u), (iidx_v, eidx_i)):
            r = idx_v[pl.ds(j * 16, 16)]
            rc = jnp.minimum(r, _VFULL - 1)
            hi = (rc >> 7) << 10
            lo = rc & 127
            for d in range(_D):
                i, k = d // 8, d % 8
                eidx_v[d, pl.ds(j * 16, 16)] = (
                    hi + (i * (_JT * 1024) + k * 128) + lo)

    pl.loop(0, _BPW // 16)(chunk)

    copies = []
    for d in range(_D):
        copies.append(pltpu.async_copy(
            uf.at[eidx_u.at[d]], urows_v.at[d], sem_u))
        copies.append(pltpu.async_copy(
            if_.at[eidx_i.at[d]], irows_v.at[d], sem_i))
    for c in copies:
        c.wait()

    def selchunk(j):
        for idx_v, rows, tail_v in ((uidx_v, urows_v, utail_v),
                                    (iidx_v, irows_v, itail_v)):
            r = idx_v[pl.ds(j * 16, 16)]
            m = r >= _VFULL
            rt = jnp.clip(r - _VFULL, 0, 63)
            for d in range(_D):
                tl = plsc.load_gather(tail_v, [rt + d * 64])
                main = rows[d, pl.ds(j * 16, 16)]
                rows[d, pl.ds(j * 16, 16)] = jnp.where(m, tl, main)

    pl.loop(0, _BPW // 16)(selchunk)

    pltpu.sync_copy(urows_v, ue2.at[:, pl.ds(base, _BPW)])
    pltpu.sync_copy(irows_v, ie2.at[:, pl.ds(base, _BPW)])


def _build_gather():
    return pl.kernel(
        _gather_body,
        out_type=(jax.ShapeDtypeStruct((_D, _B), jnp.float32),
                  jax.ShapeDtypeStruct((_D, _B), jnp.float32)),
        mesh=plsc.VectorSubcoreMesh(core_axis_name="c", subcore_axis_name="s",
                                    num_cores=_NC, num_subcores=_NS),
        scratch_types=[
            pltpu.VMEM((_BPW,), jnp.int32),
            pltpu.VMEM((_BPW,), jnp.int32),
            pltpu.VMEM((_D, _BPW), jnp.int32),
            pltpu.VMEM((_D, _BPW), jnp.int32),
            pltpu.VMEM((_D, _BPW), jnp.float32),
            pltpu.VMEM((_D, _BPW), jnp.float32),
            pltpu.VMEM((_D * 64,), jnp.float32),
            pltpu.VMEM((_D * 64,), jnp.float32),
            pltpu.SemaphoreType.DMA,
            pltpu.SemaphoreType.DMA,
        ],
        compiler_params=pltpu.CompilerParams(use_tc_tiling_on_sc=False,
                                             needs_layout_passes=False),
    )


_BLK = 2048


def _mlp_body(ue_ref, ie_ref, w1u_ref, w1i_ref, b1_ref, w2_ref, out_ref):
    dn = (((0,), (0,)), ((), ()))
    h = lax.dot_general(w1u_ref[...], ue_ref[...], dn,
                        preferred_element_type=jnp.float32)
    h = h + lax.dot_general(w1i_ref[...], ie_ref[...], dn,
                            preferred_element_type=jnp.float32)
    h = jnp.maximum(h + b1_ref[...], 0.0)
    out_ref[...] = lax.dot_general(w2_ref[...], h, dn,
                                   preferred_element_type=jnp.float32)


def _build_mlp():
    return pl.pallas_call(
        _mlp_body,
        grid=(_B // _BLK,),
        in_specs=[
            pl.BlockSpec((_D, _BLK), lambda i: (0, i)),
            pl.BlockSpec((_D, _BLK), lambda i: (0, i)),
            pl.BlockSpec((_D, _D), lambda i: (0, 0)),
            pl.BlockSpec((_D, _D), lambda i: (0, 0)),
            pl.BlockSpec((_D, 1), lambda i: (0, 0)),
            pl.BlockSpec((_D, 1), lambda i: (0, 0)),
        ],
        out_specs=pl.BlockSpec((1, _BLK), lambda i: (0, i)),
        out_shape=jax.ShapeDtypeStruct((1, _B), jnp.float32),
    )


def kernel(x, user_table, item_table, W1, b1, W2):
    uidx = x[:, 0].astype(jnp.int32)
    iidx = x[:, 1].astype(jnp.int32)
    ut3 = user_table.T.reshape(4, 8, _V)
    it3 = item_table.T.reshape(4, 8, _V)
    uf4, if4 = _build_repack()(ut3, it3)
    uf = uf4.reshape(_FLEN)
    if_ = if4.reshape(_FLEN)
    ut_tail = user_table[_VFULL:].T.reshape(_D * 64)
    it_tail = item_table[_VFULL:].T.reshape(_D * 64)
    ue2, ie2 = _build_gather()(uidx, iidx, uf, if_, ut_tail, it_tail)
    out_t = _build_mlp()(ue2, ie2, W1[:_D], W1[_D:], b1.reshape(_D, 1), W2)
    return (out_t.T, ue2.T, ie2.T)


# speedup vs baseline: 1.0011x; 1.0011x over previous
import jax
import jax.numpy as jnp
from jax import lax
from jax.experimental import pallas as pl
from jax.experimental.pallas import tpu as pltpu
from jax.experimental.pallas import tpu_sc as plsc

_B = 16384
_D = 32
_V = 1_000_000
_NC = 2
_NS = 16
_NW = _NC * _NS
_BPW = _B // _NW
_JT = 7812
_VFULL = _JT * 128
_NTILE = 4 * _JT
_TPW = _NTILE // 16
_FLEN = _NTILE * 1024


def _repack_body(ut3, it3, uf, if_, sem):
    wid = lax.axis_index("s") * _NC + lax.axis_index("c")
    t = wid % 2
    seg = wid // 2

    for tt, src, dst in ((0, ut3, uf), (1, it3, if_)):
        @pl.when(t == tt)
        def _(src=src, dst=dst):
            def fire(p):
                i = p // _JT
                j = p % _JT
                pltpu.async_copy(
                    src.at[i].at[:, pl.ds(pl.multiple_of(j * 128, 128), 128)],
                    dst.at[p], sem)

            def drain(p):
                i = p // _JT
                j = p % _JT
                pltpu.make_async_copy(
                    src.at[i].at[:, pl.ds(pl.multiple_of(j * 128, 128), 128)],
                    dst.at[p], sem).wait()

            pl.loop(seg * _TPW, (seg + 1) * _TPW)(fire)
            pl.loop(seg * _TPW, (seg + 1) * _TPW)(drain)


def _build_repack():
    return pl.kernel(
        _repack_body,
        out_type=(jax.ShapeDtypeStruct((_NTILE, 8, 128), jnp.float32),
                  jax.ShapeDtypeStruct((_NTILE, 8, 128), jnp.float32)),
        mesh=plsc.VectorSubcoreMesh(core_axis_name="c", subcore_axis_name="s",
                                    num_cores=_NC, num_subcores=_NS),
        scratch_types=[
            pltpu.SemaphoreType.DMA,
        ],
    )


def _gather_body(uidx_hbm, iidx_hbm, uf, if_, ut_tail, it_tail, ue2, ie2,
                 uidx_v, iidx_v, eidx_u, eidx_i, urows_v, irows_v,
                 utail_v, itail_v, sem_u, sem_i):
    wid = lax.axis_index("s") * _NC + lax.axis_index("c")
    base = wid * _BPW
    pltpu.sync_copy(uidx_hbm.at[pl.ds(base, _BPW)], uidx_v)
    pltpu.sync_copy(iidx_hbm.at[pl.ds(base, _BPW)], iidx_v)
    pltpu.sync_copy(ut_tail, utail_v)
    pltpu.sync_copy(it_tail, itail_v)

    def chunk(j):
        for idx_v, eidx_v in ((uidx_v, eidx_u), (iidx_v, eidx_i)):
            r = idx_v[pl.ds(j * 16, 16)]
            rc = jnp.minimum(r, _VFULL - 1)
            hi = (rc >> 7) << 10
            lo = rc & 127
            for d in range(_D):
                i, k = d // 8, d % 8
                eidx_v[d, pl.ds(j * 16, 16)] = (
                    hi + (i * (_JT * 1024) + k * 128) + lo)

    pl.loop(0, _BPW // 16)(chunk)

    copies = []
    for d in range(_D):
        copies.append(pltpu.async_copy(
            uf.at[eidx_u.at[d]], urows_v.at[d], sem_u))
        copies.append(pltpu.async_copy(
            if_.at[eidx_i.at[d]], irows_v.at[d], sem_i))
    for c in copies:
        c.wait()

    def selchunk(j):
        for idx_v, rows, tail_v in ((uidx_v, urows_v, utail_v),
                                    (iidx_v, irows_v, itail_v)):
            r = idx_v[pl.ds(j * 16, 16)]
            m = r >= _VFULL
            rt = jnp.clip(r - _VFULL, 0, 63)
            for d in range(_D):
                tl = plsc.load_gather(tail_v, [rt + d * 64])
                main = rows[d, pl.ds(j * 16, 16)]
                rows[d, pl.ds(j * 16, 16)] = jnp.where(m, tl, main)

    pl.loop(0, _BPW // 16)(selchunk)

    pltpu.sync_copy(urows_v, ue2.at[:, pl.ds(base, _BPW)])
    pltpu.sync_copy(irows_v, ie2.at[:, pl.ds(base, _BPW)])


def _build_gather():
    return pl.kernel(
        _gather_body,
        out_type=(jax.ShapeDtypeStruct((_D, _B), jnp.float32),
                  jax.ShapeDtypeStruct((_D, _B), jnp.float32)),
        mesh=plsc.VectorSubcoreMesh(core_axis_name="c", subcore_axis_name="s",
                                    num_cores=_NC, num_subcores=_NS),
        scratch_types=[
            pltpu.VMEM((_BPW,), jnp.int32),
            pltpu.VMEM((_BPW,), jnp.int32),
            pltpu.VMEM((_D, _BPW), jnp.int32),
            pltpu.VMEM((_D, _BPW), jnp.int32),
            pltpu.VMEM((_D, _BPW), jnp.float32),
            pltpu.VMEM((_D, _BPW), jnp.float32),
            pltpu.VMEM((_D * 64,), jnp.float32),
            pltpu.VMEM((_D * 64,), jnp.float32),
            pltpu.SemaphoreType.DMA,
            pltpu.SemaphoreType.DMA,
        ],
        compiler_params=pltpu.CompilerParams(use_tc_tiling_on_sc=False,
                                             needs_layout_passes=False),
    )


_BLK = 2048


def _mlp_body(ue_ref, ie_ref, w1u_ref, w1i_ref, b1_ref, w2_ref, out_ref):
    dn = (((0,), (0,)), ((), ()))
    h = lax.dot_general(w1u_ref[...], ue_ref[...], dn,
                        preferred_element_type=jnp.float32)
    h = h + lax.dot_general(w1i_ref[...], ie_ref[...], dn,
                            preferred_element_type=jnp.float32)
    h = jnp.maximum(h + b1_ref[...], 0.0)
    out_ref[...] = lax.dot_general(w2_ref[...], h, dn,
                                   preferred_element_type=jnp.float32)


def _build_mlp():
    return pl.pallas_call(
        _mlp_body,
        grid=(_B // _BLK,),
        in_specs=[
            pl.BlockSpec((_D, _BLK), lambda i: (0, i)),
            pl.BlockSpec((_D, _BLK), lambda i: (0, i)),
            pl.BlockSpec((_D, _D), lambda i: (0, 0)),
            pl.BlockSpec((_D, _D), lambda i: (0, 0)),
            pl.BlockSpec((_D, 1), lambda i: (0, 0)),
            pl.BlockSpec((_D, 1), lambda i: (0, 0)),
        ],
        out_specs=pl.BlockSpec((1, _BLK), lambda i: (0, i)),
        out_shape=jax.ShapeDtypeStruct((1, _B), jnp.float32),
    )


def kernel(x, user_table, item_table, W1, b1, W2):
    uidx = x[:, 0].astype(jnp.int32)
    iidx = x[:, 1].astype(jnp.int32)
    ut3 = user_table.T.reshape(4, 8, _V)
    it3 = item_table.T.reshape(4, 8, _V)
    uf4, if4 = _build_repack()(ut3, it3)
    uf = uf4.reshape(_FLEN)
    if_ = if4.reshape(_FLEN)
    ut_tail = user_table[_VFULL:].T.reshape(_D * 64)
    it_tail = item_table[_VFULL:].T.reshape(_D * 64)
    ue2, ie2 = _build_gather()(uidx, iidx, uf, if_, ut_tail, it_tail)
    out_t = _build_mlp()(ue2, ie2, W1[:_D], W1[_D:], b1.reshape(_D, 1), W2)
    return (out_t.T, ue2.T, ie2.T)


# speedup vs baseline: 31.6032x; 31.5674x over previous
import jax
import jax.numpy as jnp
from jax import lax
from jax.experimental import pallas as pl
from jax.experimental.pallas import tpu as pltpu
from jax.experimental.pallas import tpu_sc as plsc

_B = 16384
_D = 32
_V = 1_000_000
_NC = 2
_NS = 16
_NW = _NC * _NS
_BPW = _B // _NW
_JT = 7812
_VFULL = _JT * 128
_NTILE = 4 * _JT
_TPW = _NTILE // 16
_FLEN = _NTILE * 1024


def _flat_tiles(table):
    t3 = table.T.reshape(4, 8, _V)
    return (t3[:, :, :_VFULL].reshape(4, 8, _JT, 128)
            .transpose(0, 2, 1, 3).reshape(_FLEN))


def _gather_body(uidx_hbm, iidx_hbm, uf, if_, ut_tail, it_tail, ue2, ie2,
                 uidx_v, iidx_v, eidx_u, eidx_i, urows_v, irows_v,
                 utail_v, itail_v, sem_u, sem_i):
    wid = lax.axis_index("s") * _NC + lax.axis_index("c")
    base = wid * _BPW
    pltpu.sync_copy(uidx_hbm.at[pl.ds(base, _BPW)], uidx_v)
    pltpu.sync_copy(iidx_hbm.at[pl.ds(base, _BPW)], iidx_v)
    pltpu.sync_copy(ut_tail, utail_v)
    pltpu.sync_copy(it_tail, itail_v)

    def chunk(j):
        for idx_v, eidx_v in ((uidx_v, eidx_u), (iidx_v, eidx_i)):
            r = idx_v[pl.ds(j * 16, 16)]
            rc = jnp.minimum(r, _VFULL - 1)
            hi = (rc >> 7) << 10
            lo = rc & 127
            for d in range(_D):
                i, k = d // 8, d % 8
                eidx_v[d, pl.ds(j * 16, 16)] = (
                    hi + (i * (_JT * 1024) + k * 128) + lo)

    pl.loop(0, _BPW // 16)(chunk)

    copies = []
    for d in range(_D):
        copies.append(pltpu.async_copy(
            uf.at[eidx_u.at[d]], urows_v.at[d], sem_u))
        copies.append(pltpu.async_copy(
            if_.at[eidx_i.at[d]], irows_v.at[d], sem_i))
    for c in copies:
        c.wait()

    def selchunk(j):
        for idx_v, rows, tail_v in ((uidx_v, urows_v, utail_v),
                                    (iidx_v, irows_v, itail_v)):
            r = idx_v[pl.ds(j * 16, 16)]
            m = r >= _VFULL
            rt = jnp.clip(r - _VFULL, 0, 63)
            for d in range(_D):
                tl = plsc.load_gather(tail_v, [rt + d * 64])
                main = rows[d, pl.ds(j * 16, 16)]
                rows[d, pl.ds(j * 16, 16)] = jnp.where(m, tl, main)

    pl.loop(0, _BPW // 16)(selchunk)

    pltpu.sync_copy(urows_v, ue2.at[:, pl.ds(base, _BPW)])
    pltpu.sync_copy(irows_v, ie2.at[:, pl.ds(base, _BPW)])


def _build_gather():
    return pl.kernel(
        _gather_body,
        out_type=(jax.ShapeDtypeStruct((_D, _B), jnp.float32),
                  jax.ShapeDtypeStruct((_D, _B), jnp.float32)),
        mesh=plsc.VectorSubcoreMesh(core_axis_name="c", subcore_axis_name="s",
                                    num_cores=_NC, num_subcores=_NS),
        scratch_types=[
            pltpu.VMEM((_BPW,), jnp.int32),
            pltpu.VMEM((_BPW,), jnp.int32),
            pltpu.VMEM((_D, _BPW), jnp.int32),
            pltpu.VMEM((_D, _BPW), jnp.int32),
            pltpu.VMEM((_D, _BPW), jnp.float32),
            pltpu.VMEM((_D, _BPW), jnp.float32),
            pltpu.VMEM((_D * 64,), jnp.float32),
            pltpu.VMEM((_D * 64,), jnp.float32),
            pltpu.SemaphoreType.DMA,
            pltpu.SemaphoreType.DMA,
        ],
        compiler_params=pltpu.CompilerParams(use_tc_tiling_on_sc=False,
                                             needs_layout_passes=False),
    )


_BLK = 2048


def _mlp_body(ue_ref, ie_ref, w1u_ref, w1i_ref, b1_ref, w2_ref, out_ref):
    dn = (((0,), (0,)), ((), ()))
    h = lax.dot_general(w1u_ref[...], ue_ref[...], dn,
                        preferred_element_type=jnp.float32)
    h = h + lax.dot_general(w1i_ref[...], ie_ref[...], dn,
                            preferred_element_type=jnp.float32)
    h = jnp.maximum(h + b1_ref[...], 0.0)
    out_ref[...] = lax.dot_general(w2_ref[...], h, dn,
                                   preferred_element_type=jnp.float32)


def _build_mlp():
    return pl.pallas_call(
        _mlp_body,
        grid=(_B // _BLK,),
        in_specs=[
            pl.BlockSpec((_D, _BLK), lambda i: (0, i)),
            pl.BlockSpec((_D, _BLK), lambda i: (0, i)),
            pl.BlockSpec((_D, _D), lambda i: (0, 0)),
            pl.BlockSpec((_D, _D), lambda i: (0, 0)),
            pl.BlockSpec((_D, 1), lambda i: (0, 0)),
            pl.BlockSpec((_D, 1), lambda i: (0, 0)),
        ],
        out_specs=pl.BlockSpec((1, _BLK), lambda i: (0, i)),
        out_shape=jax.ShapeDtypeStruct((1, _B), jnp.float32),
    )


def kernel(x, user_table, item_table, W1, b1, W2):
    uidx = x[:, 0].astype(jnp.int32)
    iidx = x[:, 1].astype(jnp.int32)
    uf = _flat_tiles(user_table)
    if_ = _flat_tiles(item_table)
    ut_tail = user_table[_VFULL:].T.reshape(_D * 64)
    it_tail = item_table[_VFULL:].T.reshape(_D * 64)
    ue2, ie2 = _build_gather()(uidx, iidx, uf, if_, ut_tail, it_tail)
    out_t = _build_mlp()(ue2, ie2, W1[:_D], W1[_D:], b1.reshape(_D, 1), W2)
    return (out_t.T, ue2.T, ie2.T)
